# Initial kernel scaffold; baseline (speedup 1.0000x reference)
import jax
import jax.numpy as jnp
from jax.experimental import pallas as pl


def kernel(grid, table, gamma, beta):
    raise NotImplementedError("write your pallas kernel here")


# speedup vs baseline: 2.3692x; 2.3692x over previous
import jax
import jax.numpy as jnp
from jax import lax
from jax.experimental import pallas as pl

_EPS = 1e-5
_ROWS_PER_BLOCK = 512


def _ln_body(t_ref, g_ref, b_ref, o_ref):
    t = t_ref[...]
    mean = jnp.mean(t, axis=1, keepdims=True)
    var = jnp.mean((t - mean) ** 2, axis=1, keepdims=True)
    o_ref[...] = (t - mean) * lax.rsqrt(var + _EPS) * g_ref[...] + b_ref[...]


def _gather_body(idx_ref, nt_ref, o_ref):
    r = idx_ref.shape[1]
    ids = jnp.broadcast_to(idx_ref[0], (r, 128))
    colors = lax.broadcasted_iota(jnp.int32, (r, 128), 1)
    onehot = (ids == colors).astype(jnp.float32)
    o_ref[...] = jnp.dot(onehot, nt_ref[...], preferred_element_type=jnp.float32)


def kernel(grid, table, gamma, beta):
    batch, h, w = grid.shape
    v, d = table.shape
    n = batch * h * w

    tpad = jnp.zeros((d, d), jnp.float32).at[:v].set(table.astype(jnp.float32))
    nt = pl.pallas_call(
        _ln_body,
        out_shape=jax.ShapeDtypeStruct((d, d), jnp.float32),
    )(tpad, gamma.reshape(1, d), beta.reshape(1, d))

    r = _ROWS_PER_BLOCK
    nb = n // r
    idx3 = grid.reshape(nb, r, 1).astype(jnp.int32)

    out = pl.pallas_call(
        _gather_body,
        grid=(nb,),
        in_specs=[
            pl.BlockSpec((1, r, 1), lambda i: (i, 0, 0)),
            pl.BlockSpec((d, d), lambda i: (0, 0)),
        ],
        out_specs=pl.BlockSpec((r, d), lambda i: (i, 0)),
        out_shape=jax.ShapeDtypeStruct((n, d), jnp.float32),
    )(idx3, nt)

    return out.reshape(batch, h * w, d)


# speedup vs baseline: 16.4224x; 6.9316x over previous
import jax
import jax.numpy as jnp
from jax import lax
from jax.experimental import pallas as pl

_EPS = 1e-5
_LANE = 512
_SUB = 8
_ROWS_PER_BLOCK = _LANE * _SUB


def _ln_body(t_ref, g_ref, b_ref, o_ref):
    t = t_ref[...]
    mean = jnp.mean(t, axis=1, keepdims=True)
    var = jnp.mean((t - mean) ** 2, axis=1, keepdims=True)
    o_ref[...] = (t - mean) * lax.rsqrt(var + _EPS) * g_ref[...] + b_ref[...]


def _gather_body(idx_ref, nt_ref, o_ref):
    nt = nt_ref[...]
    colors = lax.broadcasted_iota(jnp.int32, (128, _LANE), 0)
    for j in range(_SUB):
        ids = jnp.broadcast_to(idx_ref[0, j:j + 1, :], (128, _LANE))
        onehot_t = (ids == colors).astype(jnp.float32)
        o_ref[pl.ds(j * _LANE, _LANE), :] = lax.dot_general(
            onehot_t, nt,
            dimension_numbers=(((0,), (0,)), ((), ())),
            preferred_element_type=jnp.float32,
        )


def kernel(grid, table, gamma, beta):
    batch, h, w = grid.shape
    v, d = table.shape
    n = batch * h * w

    tpad = jnp.zeros((d, d), jnp.float32).at[:v].set(table.astype(jnp.float32))
    nt = pl.pallas_call(
        _ln_body,
        out_shape=jax.ShapeDtypeStruct((d, d), jnp.float32),
    )(tpad, gamma.reshape(1, d), beta.reshape(1, d))

    nb = n // _ROWS_PER_BLOCK
    idx3 = grid.reshape(nb, _SUB, _LANE).astype(jnp.int32)

    out = pl.pallas_call(
        _gather_body,
        grid=(nb,),
        in_specs=[
            pl.BlockSpec((1, _SUB, _LANE), lambda i: (i, 0, 0)),
            pl.BlockSpec((d, d), lambda i: (0, 0)),
        ],
        out_specs=pl.BlockSpec((_ROWS_PER_BLOCK, d), lambda i: (i, 0)),
        out_shape=jax.ShapeDtypeStruct((n, d), jnp.float32),
    )(idx3, nt)

    return out.reshape(batch, h * w, d)
